# Initial kernel scaffold; baseline (speedup 1.0000x reference)
import jax
import jax.numpy as jnp
from jax.experimental import pallas as pl


def kernel(updates, mask):
    raise NotImplementedError("write your pallas kernel here")


# speedup vs baseline: 13.9809x; 13.9809x over previous
import jax
import jax.numpy as jnp
from jax import lax
from jax.experimental import pallas as pl
from jax.experimental.pallas import tpu as pltpu
from jax.experimental.pallas import tpu_sc as plsc

_B, _H, _W, _C = 4, 112, 112, 192
_HW = _H * _W
_HWO = _HW * 4
_CB = 16
_NT = 16
_RPT = _HW // _NT
_NIDX = _RPT * _CB // 128
_ACC = _HWO * _CB
_WBR = _HWO // _NT
_WCH = _WBR // 8
_INV_C = float(1.0 / _C)


def _body(u_hbm, m_hbm, out_hbm, acc, upd_v, mask_v, idx2, val1, zbuf, w1, w2):
    cid = lax.axis_index("c")
    sid = lax.axis_index("s")
    lanes = lax.iota(jnp.int32, 16)
    zeros16 = jnp.zeros((16,), jnp.float32)

    def _fill_z(i, carry):
        zbuf[pl.ds(i * 16, 16)] = zeros16
        return carry

    lax.fori_loop(0, (_WCH * 16) // 16, _fill_z, 0)

    def _round(rnd, carry):
        b = rnd // 6
        blk = rnd - b * 6
        c0 = (cid * 6 + blk) * _CB
        row0 = b * _HW + sid * _RPT

        plsc.subcore_barrier()

        def _zero(i, carry):
            pltpu.sync_copy(
                zbuf, acc.at[pl.ds(sid * _WBR * _CB + i * _WCH * 16, _WCH * 16)]
            )
            return carry

        lax.fori_loop(0, 8, _zero, 0)

        pltpu.sync_copy(u_hbm.at[pl.ds(row0, _RPT), pl.ds(c0, _CB)], upd_v)
        pltpu.sync_copy(m_hbm.at[pl.ds(row0, _RPT), pl.ds(c0, _CB)], mask_v)

        def _prep(k, carry):
            m = mask_v[k, :]
            q = m.astype(jnp.float32) * _INV_C + 0.5
            p0 = q.astype(jnp.int32)
            r = m - p0 * _C
            p = p0 + jnp.where(r < 0, -1, 0)
            idx = (p << 4) + lanes
            idx2[k >> 3, pl.ds((k & 7) * 16, 16)] = idx
            val1[pl.ds(k * 16, 16)] = upd_v[k, :]
            return carry

        lax.fori_loop(0, _RPT, _prep, 0)

        plsc.subcore_barrier()

        def _scat(j, carry):
            pltpu.sync_copy(
                val1.at[pl.ds(j * 128, 128)], acc.at[idx2.at[j]], add=True
            )
            return carry

        lax.fori_loop(0, _NIDX, _scat, 0)

        plsc.subcore_barrier()

        def _wb(i, carry):
            pltpu.sync_copy(
                acc.at[pl.ds((sid * _WBR + i * _WCH) * _CB, _WCH * _CB)], w1
            )

            def _relayout(r, carry2):
                w2[r, :] = w1[pl.ds(r * 16, 16)]
                return carry2

            lax.fori_loop(0, _WCH, _relayout, 0)
            pltpu.sync_copy(
                w2,
                out_hbm.at[
                    pl.ds(b * _HWO + sid * _WBR + i * _WCH, _WCH), pl.ds(c0, _CB)
                ],
            )
            return carry

        lax.fori_loop(0, 8, _wb, 0)
        return carry

    lax.fori_loop(0, 24, _round, 0)


@jax.jit
def _unpool(u2, m2):
    mesh = plsc.VectorSubcoreMesh(core_axis_name="c", subcore_axis_name="s")
    return pl.kernel(
        _body,
        out_type=jax.ShapeDtypeStruct((_B * _HWO, _C), jnp.float32),
        mesh=mesh,
        compiler_params=pltpu.CompilerParams(use_tc_tiling_on_sc=False),
        scratch_types=[
            pltpu.VMEM_SHARED((_ACC,), jnp.float32),
            pltpu.VMEM((_RPT, _CB), jnp.float32),
            pltpu.VMEM((_RPT, _CB), jnp.int32),
            pltpu.VMEM((_NIDX, 128), jnp.int32),
            pltpu.VMEM((_RPT * _CB,), jnp.float32),
            pltpu.VMEM((_WCH * 16,), jnp.float32),
            pltpu.VMEM((_WCH * _CB,), jnp.float32),
            pltpu.VMEM((_WCH, _CB), jnp.float32),
        ],
    )(u2, m2)


def kernel(updates, mask):
    u2 = updates.reshape(_B * _HW, _C)
    m2 = mask.astype(jnp.int32).reshape(_B * _HW, _C)
    out = _unpool(u2, m2)
    return out.reshape(_B, _H * 2, _W * 2, _C)


# speedup vs baseline: 14.9498x; 1.0693x over previous
import jax
import jax.numpy as jnp
from jax import lax
from jax.experimental import pallas as pl
from jax.experimental.pallas import tpu as pltpu
from jax.experimental.pallas import tpu_sc as plsc

_B, _H, _W, _C = 4, 112, 112, 192
_HW = _H * _W
_HWO = _HW * 4
_CB = 16
_NBLK = _C // _CB
_NT = 16
_RPT = _HW // _NT
_EPT = _RPT * _CB
_NIDX = _EPT // 128
_ACC = _HWO * _CB
_WBE = _ACC // _NT
_INV_C = float(1.0 / _C)
_WAVE = 14


def _body(u_hbm, m_hbm, out_hbm, acc, upd_v, mask_v, idx2, zbuf, sem_u, sem_m, sem_s):
    cid = lax.axis_index("c")
    sid = lax.axis_index("s")
    lanes = lax.iota(jnp.int32, 16)
    zeros16 = jnp.zeros((16,), jnp.float32)

    @plsc.parallel_loop(0, _EPT // 16, unroll=8)
    def _fill_z(i):
        zbuf[pl.ds(i * 16, 16)] = zeros16

    def _round(rnd, carry):
        b = rnd // 6
        blk = rnd - b * 6
        cb = cid * 6 + blk
        in_base = (cb * _B + b) * _HW * _CB + sid * _EPT

        cp_u = pltpu.async_copy(u_hbm.at[pl.ds(in_base, _EPT)], upd_v, sem_u)
        cp_m = pltpu.async_copy(m_hbm.at[pl.ds(in_base, _EPT)], mask_v, sem_m)

        def _zero(i, carry):
            pltpu.sync_copy(zbuf, acc.at[pl.ds(sid * _WBE + i * _EPT, _EPT)])
            return carry

        lax.fori_loop(0, 4, _zero, 0)
        cp_m.wait()

        @plsc.parallel_loop(0, _RPT, unroll=8)
        def _prep(k):
            m = mask_v[pl.ds(k * 16, 16)]
            q = m.astype(jnp.float32) * _INV_C + 0.5
            p0 = q.astype(jnp.int32)
            r = m - p0 * _C
            p = p0 + (r >> 31)
            idx = (p << 4) + lanes
            idx2[k >> 3, pl.ds((k & 7) * 16, 16)] = idx

        cp_u.wait()

        plsc.subcore_barrier()

        def _wave(w, carry):
            def _fire(j, carry2):
                pltpu.async_copy(
                    upd_v.at[pl.ds(j * 128, 128)],
                    acc.at[idx2.at[j]],
                    sem_s,
                    add=True,
                )
                return carry2

            lax.fori_loop(w * _WAVE, (w + 1) * _WAVE, _fire, 0)

            def _drain(j, carry2):
                pltpu.make_async_copy(
                    upd_v.at[pl.ds(j * 128, 128)], acc.at[idx2.at[j]], sem_s
                ).wait()
                return carry2

            lax.fori_loop(w * _WAVE, (w + 1) * _WAVE, _drain, 0)
            return carry

        lax.fori_loop(0, _NIDX // _WAVE, _wave, 0)

        plsc.subcore_barrier()

        pltpu.sync_copy(
            acc.at[pl.ds(sid * _WBE, _WBE)],
            out_hbm.at[pl.ds((cb * _B + b) * _ACC + sid * _WBE, _WBE)],
        )
        return carry

    lax.fori_loop(0, 24, _round, 0)


@jax.jit
def _unpool(u1, m1):
    mesh = plsc.VectorSubcoreMesh(core_axis_name="c", subcore_axis_name="s")
    return pl.kernel(
        _body,
        out_type=jax.ShapeDtypeStruct((_NBLK * _B * _ACC,), jnp.float32),
        mesh=mesh,
        compiler_params=pltpu.CompilerParams(use_tc_tiling_on_sc=False),
        scratch_types=[
            pltpu.VMEM_SHARED((_ACC,), jnp.float32),
            pltpu.VMEM((_EPT,), jnp.float32),
            pltpu.VMEM((_EPT,), jnp.int32),
            pltpu.VMEM((_NIDX, 128), jnp.int32),
            pltpu.VMEM((_EPT,), jnp.float32),
            pltpu.SemaphoreType.DMA,
            pltpu.SemaphoreType.DMA,
            pltpu.SemaphoreType.DMA,
        ],
    )(u1, m1)


def kernel(updates, mask):
    u1 = updates.reshape(_B * _HW, _NBLK, _CB).transpose(1, 0, 2).reshape(-1)
    m1 = (
        mask.astype(jnp.int32)
        .reshape(_B * _HW, _NBLK, _CB)
        .transpose(1, 0, 2)
        .reshape(-1)
    )
    out = _unpool(u1, m1)
    return (
        out.reshape(_NBLK, _B, _HWO, _CB)
        .transpose(1, 2, 0, 3)
        .reshape(_B, _H * 2, _W * 2, _C)
    )


# speedup vs baseline: 15.3044x; 1.0237x over previous
import jax
import jax.numpy as jnp
from jax import lax
from jax.experimental import pallas as pl
from jax.experimental.pallas import tpu as pltpu
from jax.experimental.pallas import tpu_sc as plsc

_B, _H, _W, _C = 4, 112, 112, 192
_HW = _H * _W
_NR = _B * _HW
_HWO = _HW * 4
_NRO = _B * _HWO
_CB = 16
_NBLK = _C // _CB
_NT = 16
_NCK = 2 * _HW // 128
_ACC2 = 2 * _CB * _HWO
_WBC = _HWO // 8
_INV_C = float(1.0 / _C)
_ZB = 14336
_PRB = 512
_URB = 512


def _pack_body(u_ref, m_ref, u3_ref, m3_ref):
    for cb in range(_NBLK):
        u3_ref[cb, :, :] = u_ref[:, cb * _CB : (cb + 1) * _CB].T
        m3_ref[cb, :, :] = m_ref[:, cb * _CB : (cb + 1) * _CB].T


def _unpack_body(i_ref, o_ref):
    for cb in range(_NBLK):
        o_ref[:, cb * _CB : (cb + 1) * _CB] = i_ref[cb, :, :].T


def _div_c(m):
    q = m.astype(jnp.float32) * _INV_C + 0.5
    p0 = q.astype(jnp.int32)
    r = m - p0 * _C
    return p0 + (r >> 31)


def _sc_body(
    u3, m3, o3, acc, u_b0, u_b1, m_b0, m_b1, i_b0, i_b1, zbuf,
    su0, su1, sm0, sm1, ss0, ss1,
):
    cid = lax.axis_index("c")
    sid = lax.axis_index("s")
    zeros16 = jnp.zeros((16,), jnp.float32)

    @plsc.parallel_loop(0, _ZB // 16, unroll=8)
    def _fill_z(r):
        zbuf[pl.ds(r * 16, 16)] = zeros16

    start = 12 * sid + jnp.maximum(sid - 12, 0)
    nck = 12 + (sid >= 12).astype(jnp.int32)
    region = sid >> 3
    p0_wb = _WBC * (sid & 7)

    def _chunk_coords(b, gc):
        cbl = (gc >= 98).astype(jnp.int32)
        col = b * _HW + (gc - 98 * cbl) * 128
        return cbl, col

    def _start_in(pair, b, gc, u_buf, m_buf, sem_u, sem_m):
        cbl, col = _chunk_coords(b, gc)
        cb = pair * 2 + cbl
        cp_u = pltpu.make_async_copy(u3.at[cb, :, pl.ds(col, 128)], u_buf, sem_u)
        cp_m = pltpu.make_async_copy(m3.at[cb, :, pl.ds(col, 128)], m_buf, sem_m)
        cp_u.start()
        cp_m.start()

    def _wait_in(pair, b, gc, u_buf, m_buf, sem_u, sem_m):
        cbl, col = _chunk_coords(b, gc)
        cb = pair * 2 + cbl
        pltpu.make_async_copy(u3.at[cb, :, pl.ds(col, 128)], u_buf, sem_u).wait()
        pltpu.make_async_copy(m3.at[cb, :, pl.ds(col, 128)], m_buf, sem_m).wait()

    def _fire(i_buf, u_buf, sem_s):
        def _f(k, carry):
            pltpu.async_copy(
                u_buf.at[k], acc.at[i_buf.at[k]], sem_s, add=True
            )
            return carry

        lax.fori_loop(0, _CB, _f, 0)

    def _drain(i_buf, u_buf, sem_s):
        def _d(k, carry):
            pltpu.make_async_copy(
                u_buf.at[k], acc.at[i_buf.at[k]], sem_s
            ).wait()
            return carry

        lax.fori_loop(0, _CB, _d, 0)

    def _prep(m_buf, i_buf, rowbase):
        @plsc.parallel_loop(0, 128, unroll=8)
        def _p(k):
            ch = k >> 3
            r = k & 7
            m = m_buf[ch, pl.ds(r * 16, 16)]
            idx = _div_c(m) + (rowbase + ch * _HWO)
            i_buf[ch, pl.ds(r * 16, 16)] = idx

    def _round(rnd, carry):
        pair = cid * 3 + (rnd >> 2)
        b = rnd & 3

        def _zero(i, c2):
            pltpu.sync_copy(
                zbuf, acc.at[pl.ds(sid * (_ACC2 // _NT) + i * _ZB, _ZB)]
            )
            return c2

        lax.fori_loop(0, (_ACC2 // _NT) // _ZB, _zero, 0)

        plsc.subcore_barrier()

        _start_in(pair, b, start, u_b0, m_b0, su0, sm0)

        def _chunk(c, c2):
            gc = start + c

            @pl.when((c & 1) == 0)
            def _even():
                _wait_in(pair, b, gc, u_b0, m_b0, su0, sm0)
                cbl, _ = _chunk_coords(b, gc)
                _prep(m_b0, i_b0, cbl * (_CB * _HWO))

                @pl.when(c > 0)
                def _():
                    _drain(i_b1, u_b1, ss1)

                @pl.when(c + 1 < nck)
                def _():
                    _start_in(pair, b, gc + 1, u_b1, m_b1, su1, sm1)

                _fire(i_b0, u_b0, ss0)

            @pl.when((c & 1) == 1)
            def _odd():
                _wait_in(pair, b, gc, u_b1, m_b1, su1, sm1)
                cbl, _ = _chunk_coords(b, gc)
                _prep(m_b1, i_b1, cbl * (_CB * _HWO))
                _drain(i_b0, u_b0, ss0)

                @pl.when(c + 1 < nck)
                def _():
                    _start_in(pair, b, gc + 1, u_b0, m_b0, su0, sm0)

                _fire(i_b1, u_b1, ss1)

            return c2

        lax.fori_loop(0, nck, _chunk, 0)

        @pl.when(((nck - 1) & 1) == 0)
        def _():
            _drain(i_b0, u_b0, ss0)

        @pl.when(((nck - 1) & 1) == 1)
        def _():
            _drain(i_b1, u_b1, ss1)

        plsc.subcore_barrier()

        cb_w = pair * 2 + region

        def _wb(ch, c2):
            pltpu.async_copy(
                acc.at[pl.ds((region * _CB + ch) * _HWO + p0_wb, _WBC)],
                o3.at[cb_w, ch, pl.ds(b * _HWO + p0_wb, _WBC)],
                su0,
            )
            return c2

        lax.fori_loop(0, _CB, _wb, 0)

        def _wbd(ch, c2):
            pltpu.make_async_copy(
                acc.at[pl.ds((region * _CB + ch) * _HWO + p0_wb, _WBC)],
                o3.at[cb_w, ch, pl.ds(b * _HWO + p0_wb, _WBC)],
                su0,
            ).wait()
            return c2

        lax.fori_loop(0, _CB, _wbd, 0)
        return carry

    lax.fori_loop(0, 12, _round, 0)


@jax.jit
def _unpool(u2, m2):
    u3, m3 = pl.pallas_call(
        _pack_body,
        grid=(_NR // _PRB,),
        in_specs=[
            pl.BlockSpec((_PRB, _C), lambda rb: (rb, 0)),
            pl.BlockSpec((_PRB, _C), lambda rb: (rb, 0)),
        ],
        out_specs=[
            pl.BlockSpec((_NBLK, _CB, _PRB), lambda rb: (0, 0, rb)),
            pl.BlockSpec((_NBLK, _CB, _PRB), lambda rb: (0, 0, rb)),
        ],
        out_shape=[
            jax.ShapeDtypeStruct((_NBLK, _CB, _NR), jnp.float32),
            jax.ShapeDtypeStruct((_NBLK, _CB, _NR), jnp.int32),
        ],
    )(u2, m2)

    mesh = plsc.VectorSubcoreMesh(core_axis_name="c", subcore_axis_name="s")
    o3 = pl.kernel(
        _sc_body,
        out_type=jax.ShapeDtypeStruct((_NBLK, _CB, _NRO), jnp.float32),
        mesh=mesh,
        scratch_types=[
            pltpu.VMEM_SHARED((_ACC2,), jnp.float32),
            pltpu.VMEM((_CB, 128), jnp.float32),
            pltpu.VMEM((_CB, 128), jnp.float32),
            pltpu.VMEM((_CB, 128), jnp.int32),
            pltpu.VMEM((_CB, 128), jnp.int32),
            pltpu.VMEM((_CB, 128), jnp.int32),
            pltpu.VMEM((_CB, 128), jnp.int32),
            pltpu.VMEM((_ZB,), jnp.float32),
            pltpu.SemaphoreType.DMA,
            pltpu.SemaphoreType.DMA,
            pltpu.SemaphoreType.DMA,
            pltpu.SemaphoreType.DMA,
            pltpu.SemaphoreType.DMA,
            pltpu.SemaphoreType.DMA,
        ],
    )(u3, m3)

    return pl.pallas_call(
        _unpack_body,
        grid=(_NRO // _URB,),
        in_specs=[pl.BlockSpec((_NBLK, _CB, _URB), lambda rb: (0, 0, rb))],
        out_specs=pl.BlockSpec((_URB, _C), lambda rb: (rb, 0)),
        out_shape=jax.ShapeDtypeStruct((_NRO, _C), jnp.float32),
    )(o3)


def kernel(updates, mask):
    u2 = updates.reshape(_NR, _C)
    m2 = mask.astype(jnp.int32).reshape(_NR, _C)
    out = _unpool(u2, m2)
    return out.reshape(_B, _H * 2, _W * 2, _C)


# speedup vs baseline: 42.3136x; 2.7648x over previous
import jax
import jax.numpy as jnp
from jax import lax
from jax.experimental import pallas as pl
from jax.experimental.pallas import tpu as pltpu
from jax.experimental.pallas import tpu_sc as plsc

_B, _H, _W, _C = 4, 112, 112, 192
_HW = _H * _W
_NR = _B * _HW
_HWO = _HW * 4
_NRO = _B * _HWO
_CB = 16
_NBLK = _C // _CB
_NT = 16
_NCK = 2 * _HW // 128
_ACC2 = 2 * _CB * _HWO
_WBC = _HWO // 8
_INV_C = float(1.0 / _C)
_ZB = 14336
_PKH = 16
_PKR = _PKH * _W
_UPH = 8
_UPR = _UPH * 2 * _W


def _pack_body(u_ref, m_ref, u3_ref, m3_ref):
    u3_ref[...] = u_ref[0].reshape(_PKR, _C).T.reshape(_NBLK, _CB, _PKR)
    m3_ref[...] = m_ref[0].reshape(_PKR, _C).T.reshape(_NBLK, _CB, _PKR)


def _unpack_body(i_ref, o_ref):
    t = i_ref[...].reshape(_C, _UPR).T
    o_ref[0, :, :, :] = t.reshape(_UPH, 2 * _W, _C)


def _div_c(m):
    q = m.astype(jnp.float32) * _INV_C + 0.5
    p0 = q.astype(jnp.int32)
    r = m - p0 * _C
    return p0 + (r >> 31)


def _sc_body(
    u3, m3, o3, acc, u_b0, u_b1, m_b0, m_b1, i_b0, i_b1, zbuf,
    su0, su1, sm0, sm1, ss0, ss1,
):
    cid = lax.axis_index("c")
    sid = lax.axis_index("s")
    zeros16 = jnp.zeros((16,), jnp.float32)

    @plsc.parallel_loop(0, _ZB // 16, unroll=8)
    def _fill_z(r):
        zbuf[pl.ds(r * 16, 16)] = zeros16

    start = 12 * sid + jnp.maximum(sid - 12, 0)
    nck = 12 + (sid >= 12).astype(jnp.int32)
    region = sid >> 3
    p0_wb = _WBC * (sid & 7)

    def _chunk_coords(b, gc):
        cbl = (gc >= 98).astype(jnp.int32)
        col = b * _HW + (gc - 98 * cbl) * 128
        return cbl, col

    def _start_in(pair, b, gc, u_buf, m_buf, sem_u, sem_m):
        cbl, col = _chunk_coords(b, gc)
        cb = pair * 2 + cbl
        cp_u = pltpu.make_async_copy(u3.at[cb, :, pl.ds(col, 128)], u_buf, sem_u)
        cp_m = pltpu.make_async_copy(m3.at[cb, :, pl.ds(col, 128)], m_buf, sem_m)
        cp_u.start()
        cp_m.start()

    def _wait_in(pair, b, gc, u_buf, m_buf, sem_u, sem_m):
        cbl, col = _chunk_coords(b, gc)
        cb = pair * 2 + cbl
        pltpu.make_async_copy(u3.at[cb, :, pl.ds(col, 128)], u_buf, sem_u).wait()
        pltpu.make_async_copy(m3.at[cb, :, pl.ds(col, 128)], m_buf, sem_m).wait()

    def _fire(i_buf, u_buf, sem_s):
        def _f(k, carry):
            pltpu.async_copy(
                u_buf.at[k], acc.at[i_buf.at[k]], sem_s, add=True
            )
            return carry

        lax.fori_loop(0, _CB, _f, 0)

    def _drain(i_buf, u_buf, sem_s):
        def _d(k, carry):
            pltpu.make_async_copy(
                u_buf.at[k], acc.at[i_buf.at[k]], sem_s
            ).wait()
            return carry

        lax.fori_loop(0, _CB, _d, 0)

    def _prep(m_buf, i_buf, rowbase):
        @plsc.parallel_loop(0, 128, unroll=8)
        def _p(k):
            ch = k >> 3
            r = k & 7
            m = m_buf[ch, pl.ds(r * 16, 16)]
            idx = _div_c(m) + (rowbase + ch * _HWO)
            i_buf[ch, pl.ds(r * 16, 16)] = idx

    def _round(rnd, carry):
        pair = cid * 3 + (rnd >> 1)
        bl = rnd & 1

        def _zero(i, c2):
            pltpu.sync_copy(
                zbuf, acc.at[pl.ds(sid * (_ACC2 // _NT) + i * _ZB, _ZB)]
            )
            return c2

        lax.fori_loop(0, (_ACC2 // _NT) // _ZB, _zero, 0)

        plsc.subcore_barrier()

        _start_in(pair, bl, start, u_b0, m_b0, su0, sm0)

        def _chunk(c, c2):
            gc = start + c

            @pl.when((c & 1) == 0)
            def _even():
                _wait_in(pair, bl, gc, u_b0, m_b0, su0, sm0)
                cbl, _ = _chunk_coords(bl, gc)
                _prep(m_b0, i_b0, cbl * (_CB * _HWO))

                @pl.when(c > 0)
                def _():
                    _drain(i_b1, u_b1, ss1)

                @pl.when(c + 1 < nck)
                def _():
                    _start_in(pair, bl, gc + 1, u_b1, m_b1, su1, sm1)

                _fire(i_b0, u_b0, ss0)

            @pl.when((c & 1) == 1)
            def _odd():
                _wait_in(pair, bl, gc, u_b1, m_b1, su1, sm1)
                cbl, _ = _chunk_coords(bl, gc)
                _prep(m_b1, i_b1, cbl * (_CB * _HWO))
                _drain(i_b0, u_b0, ss0)

                @pl.when(c + 1 < nck)
                def _():
                    _start_in(pair, bl, gc + 1, u_b0, m_b0, su0, sm0)

                _fire(i_b1, u_b1, ss1)

            return c2

        lax.fori_loop(0, nck, _chunk, 0)

        @pl.when(((nck - 1) & 1) == 0)
        def _():
            _drain(i_b0, u_b0, ss0)

        @pl.when(((nck - 1) & 1) == 1)
        def _():
            _drain(i_b1, u_b1, ss1)

        plsc.subcore_barrier()

        cb_w = pair * 2 + region

        def _wb(ch, c2):
            pltpu.async_copy(
                acc.at[pl.ds((region * _CB + ch) * _HWO + p0_wb, _WBC)],
                o3.at[cb_w, ch, pl.ds(bl * _HWO + p0_wb, _WBC)],
                su0,
            )
            return c2

        lax.fori_loop(0, _CB, _wb, 0)

        def _wbd(ch, c2):
            pltpu.make_async_copy(
                acc.at[pl.ds((region * _CB + ch) * _HWO + p0_wb, _WBC)],
                o3.at[cb_w, ch, pl.ds(bl * _HWO + p0_wb, _WBC)],
                su0,
            ).wait()
            return c2

        lax.fori_loop(0, _CB, _wbd, 0)
        return carry

    lax.fori_loop(0, 6, _round, 0)


import functools


def _make_sc():
    mesh = plsc.VectorSubcoreMesh(core_axis_name="c", subcore_axis_name="s")
    return pl.kernel(
        _sc_body,
        out_type=jax.ShapeDtypeStruct((_NBLK, _CB, 2 * _HWO), jnp.float32),
        mesh=mesh,
        scratch_types=[
            pltpu.VMEM_SHARED((_ACC2,), jnp.float32),
            pltpu.VMEM((_CB, 128), jnp.float32),
            pltpu.VMEM((_CB, 128), jnp.float32),
            pltpu.VMEM((_CB, 128), jnp.int32),
            pltpu.VMEM((_CB, 128), jnp.int32),
            pltpu.VMEM((_CB, 128), jnp.int32),
            pltpu.VMEM((_CB, 128), jnp.int32),
            pltpu.VMEM((_ZB,), jnp.float32),
            pltpu.SemaphoreType.DMA,
            pltpu.SemaphoreType.DMA,
            pltpu.SemaphoreType.DMA,
            pltpu.SemaphoreType.DMA,
            pltpu.SemaphoreType.DMA,
            pltpu.SemaphoreType.DMA,
        ],
    )


def _pack_half(u4, m4, b0):
    return pl.pallas_call(
        _pack_body,
        grid=(2, _HW // _PKR),
        in_specs=[
            pl.BlockSpec((1, _PKH, _W, _C), lambda b, hb: (b0 + b, hb, 0, 0)),
            pl.BlockSpec((1, _PKH, _W, _C), lambda b, hb: (b0 + b, hb, 0, 0)),
        ],
        out_specs=[
            pl.BlockSpec(
                (_NBLK, _CB, _PKR), lambda b, hb: (0, 0, b * (_HW // _PKR) + hb)
            ),
            pl.BlockSpec(
                (_NBLK, _CB, _PKR), lambda b, hb: (0, 0, b * (_HW // _PKR) + hb)
            ),
        ],
        out_shape=[
            jax.ShapeDtypeStruct((_NBLK, _CB, 2 * _HW), jnp.float32),
            jax.ShapeDtypeStruct((_NBLK, _CB, 2 * _HW), jnp.int32),
        ],
    )(u4, m4)


def _unpack_half(o3, b0, prev=None):
    kwargs = {}
    args = [o3]
    in_specs = [
        pl.BlockSpec(
            (_NBLK, _CB, _UPR), lambda b, hb: (0, 0, b * (_HWO // _UPR) + hb)
        )
    ]
    body = _unpack_body
    if prev is not None:
        def body(i_ref, p_ref, o_ref):
            del p_ref
            _unpack_body(i_ref, o_ref)

        args.append(prev)
        in_specs.append(pl.BlockSpec(memory_space=pl.ANY))
        kwargs["input_output_aliases"] = {1: 0}
    return pl.pallas_call(
        body,
        grid=(2, _HWO // _UPR),
        in_specs=in_specs,
        out_specs=pl.BlockSpec((1, _UPH, 2 * _W, _C), lambda b, hb: (b0 + b, hb, 0, 0)),
        out_shape=jax.ShapeDtypeStruct((_B, 2 * _H, 2 * _W, _C), jnp.float32),
        **kwargs,
    )(*args)


@jax.jit
def _unpool(u4, m4):
    u3a, m3a = _pack_half(u4, m4, 0)
    u3b, m3b = _pack_half(u4, m4, 2)
    sc = _make_sc()
    o3a = sc(u3a, m3a)
    o3b = sc(u3b, m3b)
    outa = _unpack_half(o3a, 0)
    return _unpack_half(o3b, 2, prev=outa)


def kernel(updates, mask):
    return _unpool(updates, mask.astype(jnp.int32))
